# baseline (speedup 1.0000x reference)
import functools

import jax
import jax.numpy as jnp
import numpy as np
from jax import lax
from jax.experimental import pallas as pl
from jax.experimental.pallas import tpu as pltpu
from jax.experimental.pallas import tpu_sc as plsc

B = 16384
F = 26
VOCAB = 100000
D = 50
DP = 64
EPS = 1e-5

NC, NS = 2, 16
NW = NC * NS
TOTAL = B * F
PER_W = TOTAL // NW
CHUNK = 128
NCHUNK = PER_W // CHUNK
NR = F * (VOCAB + 1)


def _repack_body(tab_ref, out_ref):
    out_ref[...] = jnp.pad(tab_ref[...], ((0, 0), (0, DP - D)))


def _tc_repack(tab50):
    bm = 8192
    nb = pl.cdiv(NR, bm)
    return pl.pallas_call(
        _repack_body,
        grid=(nb,),
        in_specs=[pl.BlockSpec((bm, D), lambda i: (i, 0))],
        out_specs=pl.BlockSpec((bm, DP), lambda i: (i, 0)),
        out_shape=jax.ShapeDtypeStruct((NR, DP), jnp.float32),
    )(tab50)


def _gather_body(x_ref, offs_ref, tab_ref, out_ref,
                 idx_v, offs_v, buf0, buf1, sem0, sem1):
    wid = lax.axis_index("s") * NC + lax.axis_index("c")
    base = wid * NCHUNK

    pltpu.sync_copy(x_ref.at[pl.ds(base, NCHUNK)], idx_v)
    pltpu.sync_copy(offs_ref, offs_v)

    @pl.loop(0, NCHUNK)
    def _add(r):
        for k in range(CHUNK // 16):
            sl = pl.ds(k * 16, 16)
            idx_v[r, sl] = idx_v[r, sl] + offs_v[r, sl]

    bufs = (buf0, buf1)
    sems = (sem0, sem1)

    pltpu.async_copy(tab_ref.at[idx_v.at[0]], buf0, sem0)

    @pl.loop(0, NCHUNK, step=2)
    def _chunk(g):
        for b in range(2):
            c = g + b
            nxt = bufs[1 - b]
            nsem = sems[1 - b]

            @pl.when(c + 1 < NCHUNK)
            def _():
                pltpu.async_copy(tab_ref.at[idx_v.at[c + 1]], nxt, nsem)

            pltpu.make_async_copy(tab_ref.at[idx_v.at[c]], bufs[b], sems[b]).wait()
            pltpu.sync_copy(bufs[b], out_ref.at[pl.ds((base + c) * CHUNK, CHUNK)])


def _sc_gather(x_rows, offs_rows, tab64):
    mesh = plsc.VectorSubcoreMesh(core_axis_name="c", subcore_axis_name="s",
                                  num_cores=NC, num_subcores=NS)
    kern = pl.kernel(
        _gather_body,
        out_type=jax.ShapeDtypeStruct((TOTAL, DP), jnp.float32),
        mesh=mesh,
        scratch_types=[
            pltpu.VMEM((NCHUNK, CHUNK), jnp.int32),
            pltpu.VMEM((NCHUNK, CHUNK), jnp.int32),
            pltpu.VMEM((CHUNK, DP), jnp.float32),
            pltpu.VMEM((CHUNK, DP), jnp.float32),
            pltpu.SemaphoreType.DMA,
            pltpu.SemaphoreType.DMA,
        ],
        compiler_params=pltpu.CompilerParams(use_tc_tiling_on_sc=False),
    )
    return kern(x_rows, offs_rows, tab64)


def _stats_body(raw_ref, out_ref):
    i = pl.program_id(0)

    @pl.when(i == 0)
    def _():
        out_ref[...] = jnp.zeros_like(out_ref)

    blk = raw_ref[...]
    s = jnp.sum(blk, axis=0, keepdims=True)
    s2 = jnp.sum(blk * blk, axis=0, keepdims=True)
    out_ref[...] += jnp.concatenate([s, s2], axis=0)


def _norm_body(raw_ref, stats_ref, gamma_ref, beta_ref, out_ref):
    mean = stats_ref[0, :] * (1.0 / B)
    var = stats_ref[1, :] * (1.0 / B) - mean * mean
    scale = gamma_ref[...] * lax.rsqrt(var + EPS)
    shift = beta_ref[...] - mean * scale
    normed = raw_ref[...] * scale[None, :] + shift[None, :]
    parts = [normed[:, f * DP:f * DP + D] for f in range(F)]
    out_ref[...] = jnp.concatenate(parts, axis=1)


def _tc_batchnorm(raw, gamma_p, beta_p):
    bm = 512
    nb = B // bm
    wp = F * DP
    stats = pl.pallas_call(
        _stats_body,
        grid=(nb,),
        in_specs=[pl.BlockSpec((bm, wp), lambda i: (i, 0))],
        out_specs=pl.BlockSpec((2, wp), lambda i: (0, 0)),
        out_shape=jax.ShapeDtypeStruct((2, wp), jnp.float32),
    )(raw)
    out = pl.pallas_call(
        _norm_body,
        grid=(nb,),
        in_specs=[
            pl.BlockSpec((bm, wp), lambda i: (i, 0)),
            pl.BlockSpec((2, wp), lambda i: (0, 0)),
            pl.BlockSpec((wp,), lambda i: (0,)),
            pl.BlockSpec((wp,), lambda i: (0,)),
        ],
        out_specs=pl.BlockSpec((bm, F * D), lambda i: (i, 0)),
        out_shape=jax.ShapeDtypeStruct((B, F * D), jnp.float32),
    )(raw, stats, gamma_p, beta_p)
    return out


def kernel(x, tables, gamma, beta):
    tab50 = tables.reshape(NR, D)
    tab64 = _tc_repack(tab50)

    x_rows = x.reshape(TOTAL // CHUNK, CHUNK)
    offs = np.tile(np.arange(F, dtype=np.int32) * (VOCAB + 1), PER_W // F)
    offs_rows = jnp.asarray(offs.reshape(NCHUNK, CHUNK))

    raw64 = _sc_gather(x_rows, offs_rows, tab64)

    gamma_p = jnp.pad(gamma.reshape(F, D), ((0, 0), (0, DP - D))).reshape(F * DP)
    beta_p = jnp.pad(beta.reshape(F, D), ((0, 0), (0, DP - D))).reshape(F * DP)
    return _tc_batchnorm(raw64.reshape(B, F * DP), gamma_p, beta_p)


# speedup vs baseline: 2.1354x; 2.1354x over previous
import functools

import jax
import jax.numpy as jnp
import numpy as np
from jax import lax
from jax.experimental import pallas as pl
from jax.experimental.pallas import tpu as pltpu
from jax.experimental.pallas import tpu_sc as plsc

B = 16384
F = 26
VOCAB = 100000
D = 50
DP = 64
EPS = 1e-5

NC, NS = 2, 16
NW = NC * NS
TOTAL = B * F
PER_W = TOTAL // NW
CHUNK = 128
NCHUNK = PER_W // CHUNK

HALF = 50008
RB = 7144


def _repack_body(lo_ref, hi_ref, out_ref):
    lo = lo_ref[0]
    hi = hi_ref[0]
    z = jnp.zeros((RB, DP - D), jnp.float32)
    out_ref[0] = jnp.concatenate([lo, z, hi, z], axis=-1)


def _tc_repack(tables):
    return pl.pallas_call(
        _repack_body,
        grid=(F, HALF // RB),
        in_specs=[
            pl.BlockSpec((1, RB, D), lambda f, j: (f, j, 0)),
            pl.BlockSpec((1, RB, D), lambda f, j: (f, j + HALF // RB, 0)),
        ],
        out_specs=pl.BlockSpec((1, RB, 128), lambda f, j: (f, j, 0)),
        out_shape=jax.ShapeDtypeStruct((F, HALF, 128), jnp.float32),
    )(tables, tables)


def _gather_body(x_ref, offs_ref, tab_ref, out_ref,
                 idx_v, offs_v, buf0, buf1, sem0, sem1):
    wid = lax.axis_index("s") * NC + lax.axis_index("c")
    base = wid * NCHUNK

    pltpu.sync_copy(x_ref.at[pl.ds(base, NCHUNK)], idx_v)
    pltpu.sync_copy(offs_ref, offs_v)

    @pl.loop(0, NCHUNK)
    def _add(r):
        for k in range(CHUNK // 16):
            sl = pl.ds(k * 16, 16)
            v = idx_v[r, sl]
            m = ((v - HALF) >> 31) + 1
            vm = v - m * HALF
            idx_v[r, sl] = offs_v[r, sl] + vm * 2 + m

    bufs = (buf0, buf1)
    sems = (sem0, sem1)

    pltpu.async_copy(tab_ref.at[idx_v.at[0]], buf0, sem0)

    @pl.loop(0, NCHUNK, step=2)
    def _chunk(g):
        for b in range(2):
            c = g + b

            @pl.when(c + 1 < NCHUNK)
            def _():
                pltpu.async_copy(tab_ref.at[idx_v.at[c + 1]], bufs[1 - b],
                                 sems[1 - b])

            pltpu.make_async_copy(tab_ref.at[idx_v.at[c]], bufs[b],
                                  sems[b]).wait()
            pltpu.sync_copy(bufs[b], out_ref.at[pl.ds((base + c) * CHUNK, CHUNK)])


def _sc_gather(x_rows, offs_rows, tabp):
    mesh = plsc.VectorSubcoreMesh(core_axis_name="c", subcore_axis_name="s",
                                  num_cores=NC, num_subcores=NS)
    kern = pl.kernel(
        _gather_body,
        out_type=jax.ShapeDtypeStruct((TOTAL, DP), jnp.float32),
        mesh=mesh,
        scratch_types=[
            pltpu.VMEM((NCHUNK, CHUNK), jnp.int32),
            pltpu.VMEM((NCHUNK, CHUNK), jnp.int32),
            pltpu.VMEM((CHUNK, DP), jnp.float32),
            pltpu.VMEM((CHUNK, DP), jnp.float32),
            pltpu.SemaphoreType.DMA,
            pltpu.SemaphoreType.DMA,
        ],
        compiler_params=pltpu.CompilerParams(use_tc_tiling_on_sc=False),
    )
    return kern(x_rows, offs_rows, tabp)


def _stats_body(raw_ref, out_ref):
    i = pl.program_id(0)

    @pl.when(i == 0)
    def _():
        out_ref[...] = jnp.zeros_like(out_ref)

    blk = raw_ref[...]
    s = jnp.sum(blk, axis=0, keepdims=True)
    s2 = jnp.sum(blk * blk, axis=0, keepdims=True)
    out_ref[...] += jnp.concatenate([s, s2], axis=0)


def _norm_body(raw_ref, stats_ref, gamma_ref, beta_ref, out_ref):
    mean = stats_ref[0, :] * (1.0 / B)
    var = stats_ref[1, :] * (1.0 / B) - mean * mean
    scale = gamma_ref[...] * lax.rsqrt(var + EPS)
    shift = beta_ref[...] - mean * scale
    normed = raw_ref[...] * scale[None, :] + shift[None, :]
    parts = [normed[:, f * DP:f * DP + D] for f in range(F)]
    out_ref[...] = jnp.concatenate(parts, axis=1)


def _tc_batchnorm(raw, gamma_p, beta_p):
    bm = 512
    nb = B // bm
    wp = F * DP
    stats = pl.pallas_call(
        _stats_body,
        grid=(nb,),
        in_specs=[pl.BlockSpec((bm, wp), lambda i: (i, 0))],
        out_specs=pl.BlockSpec((2, wp), lambda i: (0, 0)),
        out_shape=jax.ShapeDtypeStruct((2, wp), jnp.float32),
    )(raw)
    out = pl.pallas_call(
        _norm_body,
        grid=(nb,),
        in_specs=[
            pl.BlockSpec((bm, wp), lambda i: (i, 0)),
            pl.BlockSpec((2, wp), lambda i: (0, 0)),
            pl.BlockSpec((wp,), lambda i: (0,)),
            pl.BlockSpec((wp,), lambda i: (0,)),
        ],
        out_specs=pl.BlockSpec((bm, F * D), lambda i: (i, 0)),
        out_shape=jax.ShapeDtypeStruct((B, F * D), jnp.float32),
    )(raw, stats, gamma_p, beta_p)
    return out


def kernel(x, tables, gamma, beta):
    tabp = _tc_repack(tables).reshape(2 * F * HALF, DP)

    x_rows = x.reshape(TOTAL // CHUNK, CHUNK)
    offs = np.tile(np.arange(F, dtype=np.int32) * (2 * HALF), PER_W // F)
    offs_rows = jnp.asarray(offs.reshape(NCHUNK, CHUNK))

    raw = _sc_gather(x_rows, offs_rows, tabp)

    gamma_p = jnp.pad(gamma.reshape(F, D), ((0, 0), (0, DP - D))).reshape(F * DP)
    beta_p = jnp.pad(beta.reshape(F, D), ((0, 0), (0, DP - D))).reshape(F * DP)
    return _tc_batchnorm(raw.reshape(B, F * DP), gamma_p, beta_p)
